# Initial kernel scaffold; baseline (speedup 1.0000x reference)
import jax
import jax.numpy as jnp
from jax.experimental import pallas as pl


def kernel(h, pos, edge_index, edge_attr, W_msg1, b_msg1, W_msg2, b_msg2, W_c1, b_c1, W_c2, W_n1, b_n1, W_n2, b_n2, gamma, beta):
    raise NotImplementedError("write your pallas kernel here")


# speedup vs baseline: 2.3603x; 2.3603x over previous
import functools

import jax
import jax.numpy as jnp
from jax import lax
from jax.experimental import pallas as pl
from jax.experimental.pallas import tpu as pltpu
from jax.experimental.pallas import tpu_sc as plsc

H = 128
WPAD = H + 16
CH = 128
NC, NS = 2, 16
NW = NC * NS


def _silu(x):
    return x * jax.nn.sigmoid(x)


def _prep_body(h_ref, pos_ref, w1a_ref, w1b_ref, pa_ref, qa_ref):
    h = h_ref[...]
    pa_ref[:, :H] = jnp.dot(h, w1a_ref[...], preferred_element_type=jnp.float32)
    qa_ref[:, :H] = jnp.dot(h, w1b_ref[...], preferred_element_type=jnp.float32)
    p16 = pos_ref[...]
    pa_ref[:, H:WPAD] = -p16
    qa_ref[:, H:WPAD] = p16


def _edge_body(ga_ref, gb_ref, ea_ref, w1e_ref, wd_ref, b1_ref, w2_ref,
               b2_ref, wc1_ref, bc1_ref, wc2_ref, out_ref):
    ga = ga_ref[...]
    gb = gb_ref[...]
    pq = ga[:, :H] + gb[:, :H]
    relp = ga[:, H:WPAD] + gb[:, H:WPAD]
    d2 = jnp.sum(relp * relp, axis=1, keepdims=True)
    dist = jnp.maximum(jnp.sqrt(d2), 1e-6)
    pre = (pq + dist * wd_ref[...] + b1_ref[...]
           + jnp.dot(ea_ref[...], w1e_ref[...],
                     preferred_element_type=jnp.float32))
    x = _silu(pre)
    m = _silu(jnp.dot(x, w2_ref[...], preferred_element_type=jnp.float32)
              + b2_ref[...])
    t = _silu(jnp.dot(m, wc1_ref[...], preferred_element_type=jnp.float32)
              + bc1_ref[...])
    cw8 = jnp.dot(t, wc2_ref[...], preferred_element_type=jnp.float32)
    cw = cw8[:, 0:1]
    out_ref[:, :H] = m
    out_ref[:, H:WPAD] = relp * cw


def _node_body(h_ref, pos_ref, g0_ref, g1_ref, wn1a_ref, wn1b_ref, bn1_ref,
               wn2_ref, bn2_ref, gam_ref, bet_ref, ho_ref, po_ref):
    h = h_ref[...]
    agg = g0_ref[...] + g1_ref[...]
    h_agg = agg[:, :H]
    coord = agg[:, H:WPAD]
    u = (jnp.dot(h, wn1a_ref[...], preferred_element_type=jnp.float32)
         + jnp.dot(h_agg, wn1b_ref[...], preferred_element_type=jnp.float32)
         + bn1_ref[...])
    hn = jnp.dot(_silu(u), wn2_ref[...], preferred_element_type=jnp.float32) \
        + bn2_ref[...]
    ho = h + hn
    mu = jnp.mean(ho, axis=1, keepdims=True)
    var = jnp.mean((ho - mu) * (ho - mu), axis=1, keepdims=True)
    ho = (ho - mu) * lax.rsqrt(var + 1e-5) * gam_ref[...] + bet_ref[...]
    ho_ref[...] = ho
    po_ref[...] = pos_ref[...] + coord


def _make_gather_kernel(n_nodes, n_edges):
    nch = n_edges // CH
    mesh = plsc.VectorSubcoreMesh(core_axis_name="c", subcore_axis_name="s",
                                  num_cores=NC, num_subcores=NS)

    @functools.partial(
        pl.kernel,
        mesh=mesh,
        out_type=(jax.ShapeDtypeStruct((n_edges, WPAD), jnp.float32),
                  jax.ShapeDtypeStruct((n_edges, WPAD), jnp.float32)),
        compiler_params=pltpu.CompilerParams(use_tc_tiling_on_sc=False),
        scratch_types=[
            pltpu.VMEM((CH,), jnp.int32),
            pltpu.VMEM((CH,), jnp.int32),
            pltpu.VMEM((CH, WPAD), jnp.float32),
            pltpu.VMEM((CH, WPAD), jnp.float32),
            pltpu.SemaphoreType.DMA,
            pltpu.SemaphoreType.DMA,
        ],
    )
    def gather_k(pa_hbm, qa_hbm, row_hbm, col_hbm, ga_hbm, gb_hbm,
                 ridx_v, cidx_v, bufa, bufb, sema, semb):
        w = lax.axis_index("s") * NC + lax.axis_index("c")
        ntrips = nch // NW + jnp.where(w < (nch % NW), 1, 0)

        def body(i, carry):
            c = w + i * NW
            pltpu.sync_copy(row_hbm.at[c], ridx_v)
            pltpu.sync_copy(col_hbm.at[c], cidx_v)
            da = pltpu.async_copy(pa_hbm.at[cidx_v], bufa, sema)
            db = pltpu.async_copy(qa_hbm.at[ridx_v], bufb, semb)
            da.wait()
            db.wait()
            pltpu.sync_copy(bufa, ga_hbm.at[pl.ds(c * CH, CH)])
            pltpu.sync_copy(bufb, gb_hbm.at[pl.ds(c * CH, CH)])
            return carry

        lax.fori_loop(0, ntrips, body, 0)

    return gather_k


def _make_scatter_kernel(n_nodes, n_edges):
    nch = n_edges // CH
    rows_per = (n_nodes // NS) // 16 * 16
    mesh = plsc.VectorSubcoreMesh(core_axis_name="c", subcore_axis_name="s",
                                  num_cores=NC, num_subcores=NS)

    @functools.partial(
        pl.kernel,
        mesh=mesh,
        out_type=jax.ShapeDtypeStruct((NC, n_nodes, WPAD), jnp.float32),
        compiler_params=pltpu.CompilerParams(use_tc_tiling_on_sc=False),
        scratch_types=[
            pltpu.VMEM((CH, WPAD), jnp.float32),
            pltpu.VMEM((1, CH), jnp.int32),
            pltpu.VMEM_SHARED((n_nodes, WPAD), jnp.float32),
        ],
    )
    def scatter_k(mc_hbm, col3_hbm, zrows_hbm, out_hbm, mbuf, cidx, acc_sh):
        c = lax.axis_index("c")
        s = lax.axis_index("s")
        w = s * NC + c
        r0 = s * rows_per
        span = jnp.where(s == NS - 1, n_nodes - (NS - 1) * rows_per, rows_per)
        nz = span // 16

        def zbody(j, carry):
            base = r0 + j * 16
            pltpu.sync_copy(zrows_hbm, acc_sh.at[pl.ds(base, 16)])
            return carry

        lax.fori_loop(0, nz, zbody, 0)
        plsc.subcore_barrier()

        ntrips = nch // NW + jnp.where(w < (nch % NW), 1, 0)

        def body(i, carry):
            ck = w + i * NW
            pltpu.sync_copy(col3_hbm.at[ck], cidx)
            pltpu.sync_copy(mc_hbm.at[pl.ds(ck * CH, CH)], mbuf)
            pltpu.sync_copy(mbuf, acc_sh.at[cidx.at[0]], add=True)
            return carry

        lax.fori_loop(0, ntrips, body, 0)
        plsc.subcore_barrier()

        def obody(j, carry):
            base = r0 + j * 16
            pltpu.sync_copy(acc_sh.at[pl.ds(base, 16)],
                            out_hbm.at[c, pl.ds(base, 16)])
            return carry

        lax.fori_loop(0, nz, obody, 0)

    return scatter_k


def kernel(h, pos, edge_index, edge_attr, W_msg1, b_msg1, W_msg2, b_msg2,
           W_c1, b_c1, W_c2, W_n1, b_n1, W_n2, b_n2, gamma, beta):
    n, _ = h.shape
    e = edge_index.shape[1]
    ed = edge_attr.shape[1]
    nch = e // CH
    f32 = jnp.float32

    row = edge_index[0]
    col = edge_index[1]
    row2 = row.reshape(nch, CH)
    col2 = col.reshape(nch, CH)
    col3 = col.reshape(nch, 1, CH)
    pos16 = jnp.pad(pos, ((0, 0), (0, 16 - pos.shape[1])))
    w1a = W_msg1[:H]
    w1b = W_msg1[H:2 * H]
    wd = W_msg1[2 * H:2 * H + 1]
    w1e = W_msg1[2 * H + 1:]
    wc2p = jnp.pad(W_c2, ((0, 0), (0, 7)))
    wn1a = W_n1[:H]
    wn1b = W_n1[H:]
    b1r = b_msg1.reshape(1, H)
    b2r = b_msg2.reshape(1, H)
    bc1r = b_c1.reshape(1, H)
    bn1r = b_n1.reshape(1, H)
    bn2r = b_n2.reshape(1, H)
    gamr = gamma.reshape(1, H)
    betr = beta.reshape(1, H)
    zrows = jnp.zeros((16, WPAD), f32)

    bn = 1000
    full = lambda shp: pl.BlockSpec(shp, lambda i: (0, 0))
    pa, qa = pl.pallas_call(
        _prep_body,
        grid=(n // bn,),
        in_specs=[
            pl.BlockSpec((bn, H), lambda i: (i, 0)),
            pl.BlockSpec((bn, 16), lambda i: (i, 0)),
            full((H, H)),
            full((H, H)),
        ],
        out_specs=[
            pl.BlockSpec((bn, WPAD), lambda i: (i, 0)),
            pl.BlockSpec((bn, WPAD), lambda i: (i, 0)),
        ],
        out_shape=[
            jax.ShapeDtypeStruct((n, WPAD), f32),
            jax.ShapeDtypeStruct((n, WPAD), f32),
        ],
    )(h, pos16, w1a, w1b)

    ga, gb = _make_gather_kernel(n, e)(pa, qa, row2, col2)

    te = 2000
    mc = pl.pallas_call(
        _edge_body,
        grid=(e // te,),
        in_specs=[
            pl.BlockSpec((te, WPAD), lambda i: (i, 0)),
            pl.BlockSpec((te, WPAD), lambda i: (i, 0)),
            pl.BlockSpec((te, ed), lambda i: (i, 0)),
            full((ed, H)),
            full((1, H)),
            full((1, H)),
            full((H, H)),
            full((1, H)),
            full((H, H)),
            full((1, H)),
            full((H, 8)),
        ],
        out_specs=pl.BlockSpec((te, WPAD), lambda i: (i, 0)),
        out_shape=jax.ShapeDtypeStruct((e, WPAD), f32),
    )(ga, gb, edge_attr, w1e, wd, b1r, W_msg2, b2r, W_c1, bc1r, wc2p)

    hg = _make_scatter_kernel(n, e)(mc, col3, zrows)
    hg0 = hg[0]
    hg1 = hg[1]

    h_out, po16 = pl.pallas_call(
        _node_body,
        grid=(n // bn,),
        in_specs=[
            pl.BlockSpec((bn, H), lambda i: (i, 0)),
            pl.BlockSpec((bn, 16), lambda i: (i, 0)),
            pl.BlockSpec((bn, WPAD), lambda i: (i, 0)),
            pl.BlockSpec((bn, WPAD), lambda i: (i, 0)),
            full((H, H)),
            full((H, H)),
            full((1, H)),
            full((H, H)),
            full((1, H)),
            full((1, H)),
            full((1, H)),
        ],
        out_specs=[
            pl.BlockSpec((bn, H), lambda i: (i, 0)),
            pl.BlockSpec((bn, 16), lambda i: (i, 0)),
        ],
        out_shape=[
            jax.ShapeDtypeStruct((n, H), f32),
            jax.ShapeDtypeStruct((n, 16), f32),
        ],
    )(h, pos16, hg0, hg1, wn1a, wn1b, bn1r, W_n2, bn2r, gamr, betr)

    return (h_out, po16[:, :3])
